# Initial kernel scaffold; baseline (speedup 1.0000x reference)
import jax
import jax.numpy as jnp
from jax.experimental import pallas as pl


def kernel(data, indices, updates):
    raise NotImplementedError("write your pallas kernel here")


# speedup vs baseline: 1.5575x; 1.5575x over previous
import functools

import jax
import jax.numpy as jnp
from jax.experimental import pallas as pl
from jax.experimental.pallas import tpu as pltpu

M = 1000000
D = 64
B = 16384

COPY_BLOCK = 8000
G = 32


def _copy_body(x_ref, o_ref):
    o_ref[...] = x_ref[...]


def _scatter_body(idx_ref, dst_any, upd_ref, out_any, sem):
    step = pl.program_id(0)
    del dst_any

    def issue(i, _):
        row = idx_ref[step * G + i]
        pltpu.make_async_copy(upd_ref.at[i], out_any.at[row], sem).start()
        return 0

    jax.lax.fori_loop(0, G, issue, 0)

    def drain(i, _):
        row = idx_ref[step * G + i]
        pltpu.make_async_copy(upd_ref.at[i], out_any.at[row], sem).wait()
        return 0

    jax.lax.fori_loop(0, G, drain, 0)


@jax.jit
def kernel(data, indices, updates):
    idx = indices.reshape(B)

    copied = pl.pallas_call(
        _copy_body,
        grid=(M // COPY_BLOCK,),
        in_specs=[pl.BlockSpec((COPY_BLOCK, D), lambda i: (i, 0))],
        out_specs=pl.BlockSpec((COPY_BLOCK, D), lambda i: (i, 0)),
        out_shape=jax.ShapeDtypeStruct((M, D), jnp.float32),
    )(data)

    out = pl.pallas_call(
        _scatter_body,
        grid_spec=pltpu.PrefetchScalarGridSpec(
            num_scalar_prefetch=1,
            grid=(B // G,),
            in_specs=[
                pl.BlockSpec(memory_space=pl.ANY),
                pl.BlockSpec((G, D), lambda s, idx_ref: (s, 0)),
            ],
            out_specs=pl.BlockSpec(memory_space=pl.ANY),
            scratch_shapes=[pltpu.SemaphoreType.DMA],
        ),
        out_shape=jax.ShapeDtypeStruct((M, D), jnp.float32),
        input_output_aliases={1: 0},
    )(idx, copied, updates)

    return out
